# baseline (speedup 1.0000x reference)
import functools

import jax
import jax.numpy as jnp
from jax import lax
from jax.experimental import pallas as pl
from jax.experimental.pallas import tpu as pltpu
from jax.experimental.pallas import tpu_sc as plsc

N_DIM = 32
N_RATINGS = 5
B = 16384

_NC = 2
_NS = 16
_NW = _NC * _NS
_BPW = B // _NW
_EPW = _BPW * N_RATINGS


def _expand_idx5(idx_v, idx5_v):
    lanes = jax.lax.iota(jnp.int32, 16)
    for k in range(_EPW // 16):
        e = lanes + (16 * k)
        b = lax.div(e, N_RATINGS)
        j = e - b * N_RATINGS
        u = plsc.load_gather(idx_v, [b])
        idx5_v[pl.ds(16 * k, 16)] = u * N_RATINGS + j


def _sc_gather_body(users_hbm, items_hbm, u_emb_hbm, i_emb_hbm,
                    uint_hbm, iint_hbm,
                    ue_out, ie_out, su_out, si_out,
                    idxu_v, idxi_v, idx5u_v, idx5i_v,
                    ue_v, ie_v, su_v, si_v,
                    sem_ue, sem_ie, sem_su, sem_si):
    wid = lax.axis_index("s") * _NC + lax.axis_index("c")
    base = wid * _BPW
    pltpu.sync_copy(users_hbm.at[pl.ds(base, _BPW)], idxu_v)
    pltpu.sync_copy(items_hbm.at[pl.ds(base, _BPW)], idxi_v)
    cue = pltpu.async_copy(u_emb_hbm.at[idxu_v], ue_v, sem_ue)
    cie = pltpu.async_copy(i_emb_hbm.at[idxi_v], ie_v, sem_ie)
    _expand_idx5(idxu_v, idx5u_v)
    _expand_idx5(idxi_v, idx5i_v)
    csu = pltpu.async_copy(uint_hbm.at[idx5u_v], su_v, sem_su)
    csi = pltpu.async_copy(iint_hbm.at[idx5i_v], si_v, sem_si)
    cue.wait()
    pltpu.sync_copy(ue_v, ue_out.at[pl.ds(base, _BPW), :])
    cie.wait()
    pltpu.sync_copy(ie_v, ie_out.at[pl.ds(base, _BPW), :])
    csu.wait()
    pltpu.sync_copy(su_v, su_out.at[pl.ds(wid * _EPW, _EPW)])
    csi.wait()
    pltpu.sync_copy(si_v, si_out.at[pl.ds(wid * _EPW, _EPW)])


_sc_gather = functools.partial(
    pl.kernel,
    out_type=(
        jax.ShapeDtypeStruct((B, N_DIM), jnp.float32),
        jax.ShapeDtypeStruct((B, N_DIM), jnp.float32),
        jax.ShapeDtypeStruct((B * N_RATINGS,), jnp.float32),
        jax.ShapeDtypeStruct((B * N_RATINGS,), jnp.float32),
    ),
    mesh=plsc.VectorSubcoreMesh(core_axis_name="c", subcore_axis_name="s"),
    compiler_params=pltpu.CompilerParams(use_tc_tiling_on_sc=False,
                                         needs_layout_passes=False),
    scratch_types=[
        pltpu.VMEM((_BPW,), jnp.int32),
        pltpu.VMEM((_BPW,), jnp.int32),
        pltpu.VMEM((_EPW,), jnp.int32),
        pltpu.VMEM((_EPW,), jnp.int32),
        pltpu.VMEM((_BPW, N_DIM), jnp.float32),
        pltpu.VMEM((_BPW, N_DIM), jnp.float32),
        pltpu.VMEM((_EPW,), jnp.float32),
        pltpu.VMEM((_EPW,), jnp.float32),
        pltpu.SemaphoreType.DMA,
        pltpu.SemaphoreType.DMA,
        pltpu.SemaphoreType.DMA,
        pltpu.SemaphoreType.DMA,
    ],
)(_sc_gather_body)


def _tc_mlp_body(ue_ref, ie_ref, su_ref, si_ref,
                 w1u_ref, w1i_ref, b1_ref, w2_ref, b2_ref, out_ref):
    x1 = jnp.dot(ue_ref[...], w1u_ref[...], preferred_element_type=jnp.float32)
    x2 = jnp.dot(ie_ref[...], w1i_ref[...], preferred_element_type=jnp.float32)
    h = jnp.maximum(x1 + x2 + b1_ref[...], 0.0)
    t = jnp.dot(h, w2_ref[...], preferred_element_type=jnp.float32)
    out_ref[...] = t + b2_ref[...] + su_ref[...] + si_ref[...]


def _tc_mlp(ue, ie, su, si, w1u, w1i, b1, w2, b2):
    blk = 2048
    grid = B // blk
    return pl.pallas_call(
        _tc_mlp_body,
        out_shape=jax.ShapeDtypeStruct((B, N_RATINGS), jnp.float32),
        grid=(grid,),
        in_specs=[
            pl.BlockSpec((blk, N_DIM), lambda i: (i, 0)),
            pl.BlockSpec((blk, N_DIM), lambda i: (i, 0)),
            pl.BlockSpec((blk, N_RATINGS), lambda i: (i, 0)),
            pl.BlockSpec((blk, N_RATINGS), lambda i: (i, 0)),
            pl.BlockSpec((N_DIM, 3 * N_RATINGS), lambda i: (0, 0)),
            pl.BlockSpec((N_DIM, 3 * N_RATINGS), lambda i: (0, 0)),
            pl.BlockSpec((1, 3 * N_RATINGS), lambda i: (0, 0)),
            pl.BlockSpec((3 * N_RATINGS, N_RATINGS), lambda i: (0, 0)),
            pl.BlockSpec((1, N_RATINGS), lambda i: (0, 0)),
        ],
        out_specs=pl.BlockSpec((blk, N_RATINGS), lambda i: (i, 0)),
    )(ue, ie, su, si, w1u, w1i, b1, w2, b2)


def kernel(users, items, u_emb, i_emb, u_intercept, i_intercept,
           W1, b1, W2, b2):
    ue, ie, su, si = _sc_gather(users, items, u_emb, i_emb,
                                u_intercept.reshape(-1),
                                i_intercept.reshape(-1))
    return _tc_mlp(ue, ie,
                   su.reshape(B, N_RATINGS), si.reshape(B, N_RATINGS),
                   W1[:N_DIM], W1[N_DIM:],
                   b1.reshape(1, -1), W2, b2.reshape(1, -1))
